# baseline (speedup 1.0000x reference)
import functools
import math

import jax
import jax.numpy as jnp
from jax import lax
from jax.experimental import pallas as pl
from jax.experimental.pallas import tpu as pltpu, tpu_sc as plsc
import numpy as np

VOCAB = 100000
WORD_DIM = 128
FSA_DIM = 64
NUM_STATES = 256
B = 1024
S = 200
OUT_DIM = WORD_DIM + FSA_DIM
SCALE = math.sqrt(float(WORD_DIM))
SPAD = 208

SBLK = 8
BBLK = 256


def _pe_table(seq_len, dim):
    pos = jnp.arange(seq_len, dtype=jnp.float32)[:, None]
    div = jnp.exp(jnp.arange(0, dim, 2, dtype=jnp.float32) * (-np.log(10000.0) / dim))
    ang = pos * div[None, :]
    pe = jnp.zeros((seq_len, dim), dtype=jnp.float32)
    pe = pe.at[:, 0::2].set(jnp.sin(ang))
    pe = pe.at[:, 1::2].set(jnp.cos(ang))
    return pe


def _make_sc_kernel():
    mesh = plsc.VectorSubcoreMesh(core_axis_name="c", subcore_axis_name="s")
    info = plsc.get_sparse_core_info()
    NC, NS = info.num_cores, info.num_subcores
    NW = NC * NS
    EPW = B // NW

    @functools.partial(
        pl.kernel,
        out_type=(
            jax.ShapeDtypeStruct((B, S, WORD_DIM), jnp.float32),
            jax.ShapeDtypeStruct((S, B), jnp.int32),
        ),
        mesh=mesh,
        compiler_params=pltpu.CompilerParams(
            use_tc_tiling_on_sc=False, needs_layout_passes=False),
        scratch_types=[
            pltpu.VMEM((S, EPW), jnp.int32),
            pltpu.VMEM((SPAD, EPW), jnp.int32),
            pltpu.VMEM((SPAD,), jnp.int32),
            pltpu.VMEM((EPW,), jnp.int32),
            pltpu.VMEM((S, WORD_DIM), jnp.float32),
            pltpu.SemaphoreType.DMA,
            pltpu.SemaphoreType.DMA,
        ],
    )
    def sc_kernel(wids_t, wtab, trans, word_raw, states_t,
                  syms, hist, syms_e, idxbuf, wbuf, sem_a, sem_w):
        wid = lax.axis_index("s") * NC + lax.axis_index("c")
        base = wid * EPW
        lanes = lax.iota(jnp.int32, 16)

        pltpu.sync_copy(wids_t.at[:, pl.ds(base, EPW)], syms)

        def scan_step(t, _):
            for g in range(EPW // 16):
                sym = syms[t, pl.ds(16 * g, 16)]
                prev = hist[lax.max(t - 1, 0), pl.ds(16 * g, 16)]
                state = jnp.where(t == 0, jnp.int32(0), prev)
                idxbuf[pl.ds(16 * g, 16)] = state * jnp.int32(VOCAB) + sym
            pltpu.async_copy(trans.at[idxbuf], hist.at[t], sem_a).wait()
            return 0

        lax.fori_loop(0, S, scan_step, 0, unroll=False)

        pltpu.sync_copy(hist.at[pl.ds(0, S)], states_t.at[:, pl.ds(base, EPW)])

        def emit_example(e, _):
            for j in range(SPAD // 16):
                t_idx = lanes + (16 * j)
                e_idx = jnp.full((16,), e, dtype=jnp.int32)
                v = plsc.load_gather(syms, [jnp.minimum(t_idx, S - 1), e_idx])
                syms_e[pl.ds(16 * j, 16)] = v
            cw0 = pltpu.async_copy(
                wtab.at[syms_e.at[pl.ds(0, 128)]], wbuf.at[pl.ds(0, 128)], sem_w)
            cw1 = pltpu.async_copy(
                wtab.at[syms_e.at[pl.ds(128, S - 128)]],
                wbuf.at[pl.ds(128, S - 128)], sem_w)
            cw0.wait(); cw1.wait()
            pltpu.sync_copy(wbuf, word_raw.at[base + e])
            return 0

        lax.fori_loop(0, EPW, emit_example, 0, unroll=False)

    return sc_kernel


def _tc_assemble(word_raw, states_t, fsa_table, pe):
    grid = (B // BBLK, S // SBLK)

    def body(w_ref, st_ref, ft_ref, pe_ref, o_ref):
        o_ref[:, :, 0:WORD_DIM] = (
            w_ref[...] * jnp.float32(SCALE) + pe_ref[...][None, :, :])
        qiota = lax.broadcasted_iota(jnp.int32, (BBLK, NUM_STATES), 1)
        for i in range(SBLK):
            onehot = (qiota == st_ref[i][:, None]).astype(jnp.float32)
            fsa = lax.dot_general(onehot, ft_ref[...], (((1,), (0,)), ((), ())),
                                  precision=lax.Precision.HIGHEST,
                                  preferred_element_type=jnp.float32)
            o_ref[:, i, WORD_DIM:OUT_DIM] = fsa

    return pl.pallas_call(
        body,
        grid=grid,
        in_specs=[
            pl.BlockSpec((BBLK, SBLK, WORD_DIM), lambda b, s: (b, s, 0)),
            pl.BlockSpec((SBLK, BBLK), lambda b, s: (s, b)),
            pl.BlockSpec((NUM_STATES, FSA_DIM), lambda b, s: (0, 0)),
            pl.BlockSpec((SBLK, WORD_DIM), lambda b, s: (s, 0)),
        ],
        out_specs=pl.BlockSpec((BBLK, SBLK, OUT_DIM), lambda b, s: (b, s, 0)),
        out_shape=jax.ShapeDtypeStruct((B, S, OUT_DIM), jnp.float32),
        compiler_params=pltpu.CompilerParams(
            dimension_semantics=("arbitrary", "arbitrary")),
    )(word_raw, states_t, fsa_table, pe)


def kernel(word_id_sequence, word_table, fsa_table, fsa_transitions):
    pe = _pe_table(S, WORD_DIM)
    wids_t = word_id_sequence.T
    trans_flat = fsa_transitions.reshape(-1)
    sc = _make_sc_kernel()
    word_raw, states_t = sc(wids_t, word_table, trans_flat)
    return _tc_assemble(word_raw, states_t, fsa_table, pe)


# speedup vs baseline: 1.3293x; 1.3293x over previous
import functools
import math

import jax
import jax.numpy as jnp
from jax import lax
from jax.experimental import pallas as pl
from jax.experimental.pallas import tpu as pltpu, tpu_sc as plsc
import numpy as np

VOCAB = 100000
WORD_DIM = 128
FSA_DIM = 64
NUM_STATES = 256
B = 1024
S = 200
OUT_DIM = WORD_DIM + FSA_DIM
SCALE = math.sqrt(float(WORD_DIM))
SPAD = 208

SBLK = 8
BBLK = 512


def _pe_table(seq_len, dim):
    pos = jnp.arange(seq_len, dtype=jnp.float32)[:, None]
    div = jnp.exp(jnp.arange(0, dim, 2, dtype=jnp.float32) * (-np.log(10000.0) / dim))
    ang = pos * div[None, :]
    pe = jnp.zeros((seq_len, dim), dtype=jnp.float32)
    pe = pe.at[:, 0::2].set(jnp.sin(ang))
    pe = pe.at[:, 1::2].set(jnp.cos(ang))
    return pe


def _make_sc_kernel():
    mesh = plsc.VectorSubcoreMesh(core_axis_name="c", subcore_axis_name="s")
    info = plsc.get_sparse_core_info()
    NC, NS = info.num_cores, info.num_subcores
    NW = NC * NS
    EPW = B // NW

    @functools.partial(
        pl.kernel,
        out_type=(
            jax.ShapeDtypeStruct((B, S, WORD_DIM), jnp.float32),
            jax.ShapeDtypeStruct((S, B), jnp.int32),
        ),
        mesh=mesh,
        compiler_params=pltpu.CompilerParams(
            use_tc_tiling_on_sc=False, needs_layout_passes=False),
        scratch_types=[
            pltpu.VMEM((EPW, S), jnp.int32),
            pltpu.VMEM((SPAD, EPW), jnp.int32),
            pltpu.VMEM((EPW,), jnp.int32),
            pltpu.VMEM((2, S, WORD_DIM), jnp.float32),
            pltpu.SemaphoreType.DMA,
            pltpu.SemaphoreType.DMA,
            pltpu.SemaphoreType.DMA,
        ],
    )
    def sc_kernel(wids, wtab, trans, word_raw, states_t,
                  syms, hist, idxbuf, wbuf, sem_a, sem_w, sem_c):
        wid = lax.axis_index("s") * NC + lax.axis_index("c")
        base = wid * EPW
        lanes = lax.iota(jnp.int32, 16)

        pltpu.sync_copy(wids.at[pl.ds(base, EPW)], syms)

        def scan_step(t):
            for g in range(EPW // 16):
                row = lanes + jnp.int32(16 * g)
                col = jnp.full((16,), t, dtype=jnp.int32)
                sym = plsc.load_gather(syms, [row, col])
                if t == 0:
                    idx = sym
                else:
                    prev = hist[t - 1, pl.ds(16 * g, 16)]
                    idx = prev * jnp.int32(VOCAB) + sym
                idxbuf[pl.ds(16 * g, 16)] = idx
            pltpu.async_copy(trans.at[idxbuf], hist.at[t], sem_a).wait()

        couts = [None, None]
        t_next = 0
        for e in range(EPW):
            if couts[e % 2] is not None:
                couts[e % 2].wait()
            g0 = pltpu.async_copy(
                wtab.at[syms.at[e, pl.ds(0, 128)]],
                wbuf.at[e % 2, pl.ds(0, 128)], sem_w)
            g1 = pltpu.async_copy(
                wtab.at[syms.at[e, pl.ds(128, S - 128)]],
                wbuf.at[e % 2, pl.ds(128, S - 128)], sem_w)
            t_end = ((e + 1) * S) // EPW
            for t in range(t_next, t_end):
                scan_step(t)
            t_next = t_end
            g0.wait(); g1.wait()
            couts[e % 2] = pltpu.async_copy(
                wbuf.at[e % 2], word_raw.at[base + e], sem_c)
        couts[0].wait()
        couts[1].wait()

        pltpu.sync_copy(hist.at[pl.ds(0, S)], states_t.at[:, pl.ds(base, EPW)])

    return sc_kernel


def _tc_assemble(word_raw, states_t, fsa_table, pe):
    grid = (S // SBLK, B // BBLK)

    def body(w_ref, st_ref, ft_ref, pe_ref, o_ref):
        ident = jnp.eye(WORD_DIM, dtype=jnp.float32)
        qiota = lax.broadcasted_iota(jnp.int32, (NUM_STATES, BBLK), 0)
        for i in range(SBLK):
            w = w_ref[:, i, :]
            wt = lax.dot_general(ident, w, (((1,), (1,)), ((), ())),
                                 precision=lax.Precision.HIGHEST,
                                 preferred_element_type=jnp.float32)
            o_ref[i, 0:WORD_DIM, :] = wt * jnp.float32(SCALE) + pe_ref[i][:, None]
            onehot = (qiota == st_ref[i][None, :]).astype(jnp.float32)
            fsa = lax.dot_general(ft_ref[...], onehot, (((0,), (0,)), ((), ())),
                                  precision=lax.Precision.HIGHEST,
                                  preferred_element_type=jnp.float32)
            o_ref[i, WORD_DIM:OUT_DIM, :] = fsa

    return pl.pallas_call(
        body,
        grid=grid,
        in_specs=[
            pl.BlockSpec((BBLK, SBLK, WORD_DIM), lambda s, b: (b, s, 0)),
            pl.BlockSpec((SBLK, BBLK), lambda s, b: (s, b)),
            pl.BlockSpec((NUM_STATES, FSA_DIM), lambda s, b: (0, 0)),
            pl.BlockSpec((SBLK, WORD_DIM), lambda s, b: (s, 0)),
        ],
        out_specs=pl.BlockSpec((SBLK, OUT_DIM, BBLK), lambda s, b: (s, 0, b)),
        out_shape=jax.ShapeDtypeStruct((S, OUT_DIM, B), jnp.float32),
        compiler_params=pltpu.CompilerParams(
            dimension_semantics=("arbitrary", "arbitrary")),
    )(word_raw, states_t, fsa_table, pe)


def kernel(word_id_sequence, word_table, fsa_table, fsa_transitions):
    pe = _pe_table(S, WORD_DIM)
    trans_flat = fsa_transitions.reshape(-1)
    sc = _make_sc_kernel()
    word_raw, states_t = sc(word_id_sequence, word_table, trans_flat)
    out_sdb = _tc_assemble(word_raw, states_t, fsa_table, pe)
    return out_sdb.transpose(2, 0, 1)


# speedup vs baseline: 1.7116x; 1.2876x over previous
import functools
import math

import jax
import jax.numpy as jnp
from jax import lax
from jax.experimental import pallas as pl
from jax.experimental.pallas import tpu as pltpu, tpu_sc as plsc
import numpy as np

VOCAB = 100000
WORD_DIM = 128
FSA_DIM = 64
NUM_STATES = 256
B = 1024
S = 200
OUT_DIM = WORD_DIM + FSA_DIM
SCALE = math.sqrt(float(WORD_DIM))
SPAD = 208

SBLK = 8
BBLK = 512


def _pe_table(seq_len, dim):
    pos = jnp.arange(seq_len, dtype=jnp.float32)[:, None]
    div = jnp.exp(jnp.arange(0, dim, 2, dtype=jnp.float32) * (-np.log(10000.0) / dim))
    ang = pos * div[None, :]
    pe = jnp.zeros((seq_len, dim), dtype=jnp.float32)
    pe = pe.at[:, 0::2].set(jnp.sin(ang))
    pe = pe.at[:, 1::2].set(jnp.cos(ang))
    return pe


def _make_sc_kernel():
    mesh = plsc.VectorSubcoreMesh(core_axis_name="c", subcore_axis_name="s")
    info = plsc.get_sparse_core_info()
    NC, NS = info.num_cores, info.num_subcores
    NW = NC * NS
    EPW = B // NW

    @functools.partial(
        pl.kernel,
        out_type=(
            jax.ShapeDtypeStruct((B, S, WORD_DIM), jnp.float32),
            jax.ShapeDtypeStruct((S, B), jnp.int32),
        ),
        mesh=mesh,
        compiler_params=pltpu.CompilerParams(
            use_tc_tiling_on_sc=False, needs_layout_passes=False),
        scratch_types=[
            pltpu.VMEM((EPW, S), jnp.int32),
            pltpu.VMEM((SPAD, EPW), jnp.int32),
            pltpu.VMEM((EPW,), jnp.int32),
            pltpu.VMEM((2, S, WORD_DIM), jnp.float32),
            pltpu.SemaphoreType.DMA,
            pltpu.SemaphoreType.DMA,
            pltpu.SemaphoreType.DMA,
        ],
    )
    def sc_kernel(wids, wtab, trans, word_raw, states_t,
                  syms, hist, idxbuf, wbuf, sem_a, sem_w, sem_c):
        wid = lax.axis_index("s") * NC + lax.axis_index("c")
        base = wid * EPW
        lanes = lax.iota(jnp.int32, 16)

        pltpu.sync_copy(wids.at[pl.ds(base, EPW)], syms)

        def scan_step(t):
            for g in range(EPW // 16):
                row = lanes + jnp.int32(16 * g)
                col = jnp.full((16,), t, dtype=jnp.int32)
                sym = plsc.load_gather(syms, [row, col])
                if t == 0:
                    idx = sym
                else:
                    prev = hist[t - 1, pl.ds(16 * g, 16)]
                    idx = prev * jnp.int32(VOCAB) + sym
                idxbuf[pl.ds(16 * g, 16)] = idx
            pltpu.async_copy(trans.at[idxbuf], hist.at[t], sem_a).wait()

        couts = [None, None]
        t_next = 0
        for e in range(EPW):
            if couts[e % 2] is not None:
                couts[e % 2].wait()
            g0 = pltpu.async_copy(
                wtab.at[syms.at[e, pl.ds(0, 128)]],
                wbuf.at[e % 2, pl.ds(0, 128)], sem_w)
            g1 = pltpu.async_copy(
                wtab.at[syms.at[e, pl.ds(128, S - 128)]],
                wbuf.at[e % 2, pl.ds(128, S - 128)], sem_w)
            t_end = ((e + 1) * S) // EPW
            for t in range(t_next, t_end):
                scan_step(t)
            t_next = t_end
            g0.wait(); g1.wait()
            couts[e % 2] = pltpu.async_copy(
                wbuf.at[e % 2], word_raw.at[base + e], sem_c)
        couts[0].wait()
        couts[1].wait()

        pltpu.sync_copy(hist.at[pl.ds(0, S)], states_t.at[:, pl.ds(base, EPW)])

    return sc_kernel


def _tc_assemble(word_raw, states_t, fsa_table, pe):
    f_hi = fsa_table.astype(jnp.bfloat16)
    r1 = fsa_table - f_hi.astype(jnp.float32)
    f_mid = r1.astype(jnp.bfloat16)
    r2 = r1 - f_mid.astype(jnp.float32)
    f_lo = r2.astype(jnp.bfloat16)
    grid = (S // SBLK, B // BBLK)

    def body(w_ref, st_ref, fh_ref, fm_ref, fl_ref, pe_ref, o_ref):
        qiota = lax.broadcasted_iota(jnp.int32, (NUM_STATES, BBLK), 0)
        for i in range(SBLK):
            wt = w_ref[:, i, :].T
            o_ref[i, 0:WORD_DIM, :] = wt * jnp.float32(SCALE) + pe_ref[i][:, None]
            onehot = (qiota == st_ref[i][None, :]).astype(jnp.bfloat16)
            fsa = jnp.zeros((FSA_DIM, BBLK), jnp.float32)
            for part in (fh_ref, fm_ref, fl_ref):
                fsa = fsa + lax.dot_general(
                    part[...], onehot, (((0,), (0,)), ((), ())),
                    preferred_element_type=jnp.float32)
            o_ref[i, WORD_DIM:OUT_DIM, :] = fsa

    return pl.pallas_call(
        body,
        grid=grid,
        in_specs=[
            pl.BlockSpec((BBLK, SBLK, WORD_DIM), lambda s, b: (b, s, 0)),
            pl.BlockSpec((SBLK, BBLK), lambda s, b: (s, b)),
            pl.BlockSpec((NUM_STATES, FSA_DIM), lambda s, b: (0, 0)),
            pl.BlockSpec((NUM_STATES, FSA_DIM), lambda s, b: (0, 0)),
            pl.BlockSpec((NUM_STATES, FSA_DIM), lambda s, b: (0, 0)),
            pl.BlockSpec((SBLK, WORD_DIM), lambda s, b: (s, 0)),
        ],
        out_specs=pl.BlockSpec((SBLK, OUT_DIM, BBLK), lambda s, b: (s, 0, b)),
        out_shape=jax.ShapeDtypeStruct((S, OUT_DIM, B), jnp.float32),
        compiler_params=pltpu.CompilerParams(
            dimension_semantics=("arbitrary", "arbitrary")),
    )(word_raw, states_t, f_hi, f_mid, f_lo, pe)


def kernel(word_id_sequence, word_table, fsa_table, fsa_transitions):
    pe = _pe_table(S, WORD_DIM)
    trans_flat = fsa_transitions.reshape(-1)
    sc = _make_sc_kernel()
    word_raw, states_t = sc(word_id_sequence, word_table, trans_flat)
    out_sdb = _tc_assemble(word_raw, states_t, fsa_table, pe)
    return out_sdb.transpose(2, 0, 1)
